# Initial kernel scaffold; baseline (speedup 1.0000x reference)
import jax
import jax.numpy as jnp
from jax.experimental import pallas as pl


def kernel(hidden_states, W_router, W_gate, W_up, W_down, A_gate, B_gate, A_up, B_up, A_down, B_down):
    raise NotImplementedError("write your pallas kernel here")


# speedup vs baseline: 4.3960x; 4.3960x over previous
import jax
import jax.numpy as jnp
from jax.experimental import pallas as pl

_E = 8
_K = 2
_D = 1024
_F = 2816
_R = 16
_T = 4096
_ER = _E * _R
_TILE = 256


def _fused(x_ref, xb_ref, wr_ref, wgu_ref, agu_ref, bg_ref, bu_ref,
           adt_ref, wd_ref, bd_ref, out_ref):
    f32 = jnp.float32
    bf16 = jnp.bfloat16
    x = x_ref[...]
    xb = xb_ref[...]

    logits = jnp.dot(x, wr_ref[...], preferred_element_type=f32)
    eidx = jax.lax.broadcasted_iota(jnp.int32, (_TILE, _E), 1)
    l0 = jnp.max(logits, axis=-1, keepdims=True)
    i0 = jnp.min(jnp.where(logits == l0, eidx, _E), axis=-1, keepdims=True)
    masked = jnp.where(eidx == i0, -jnp.inf, logits)
    l1 = jnp.max(masked, axis=-1, keepdims=True)
    i1 = jnp.min(jnp.where(masked == l1, eidx, _E), axis=-1, keepdims=True)
    w0 = jax.nn.sigmoid(l0 - l1)
    w1 = 1.0 - w0

    g0u0 = jnp.dot(xb, wgu_ref[...], preferred_element_type=f32)
    g0 = g0u0[:, :_F]
    u0 = g0u0[:, _F:]
    pgu = jnp.dot(xb, agu_ref[...], preferred_element_type=f32)
    pg = pgu[:, :_ER]
    pu = pgu[:, _ER:]

    cidx = jax.lax.broadcasted_iota(jnp.int32, (_TILE, _ER), 1) // _R
    hbar = jnp.zeros((_TILE, _F), f32)
    q = jnp.zeros((_TILE, _ER), f32)
    for ik, wk in ((i0, w0), (i1, w1)):
        mk = cidx == ik
        g = g0 + jnp.dot(jnp.where(mk, pg, 0.0).astype(bf16), bg_ref[...],
                         preferred_element_type=f32)
        u = u0 + jnp.dot(jnp.where(mk, pu, 0.0).astype(bf16), bu_ref[...],
                         preferred_element_type=f32)
        wh = (g * jax.nn.sigmoid(g)) * u * wk
        hbar = hbar + wh
        qf = jnp.dot(wh.astype(bf16), adt_ref[...], preferred_element_type=f32)
        q = q + jnp.where(mk, qf, 0.0)

    out_ref[...] = (
        jnp.dot(hbar.astype(bf16), wd_ref[...], preferred_element_type=f32)
        + jnp.dot(q.astype(bf16), bd_ref[...], preferred_element_type=f32))


def kernel(hidden_states, W_router, W_gate, W_up, W_down,
           A_gate, B_gate, A_up, B_up, A_down, B_down):
    f32 = jnp.float32
    bf16 = jnp.bfloat16
    xb = hidden_states.astype(bf16)
    wgu = jnp.concatenate([W_gate, W_up], axis=1).astype(bf16)
    agu = jnp.concatenate(
        [A_gate.reshape(_ER, _D).T, A_up.reshape(_ER, _D).T],
        axis=1).astype(bf16)
    bg = B_gate.transpose(0, 2, 1).reshape(_ER, _F).astype(bf16)
    bu = B_up.transpose(0, 2, 1).reshape(_ER, _F).astype(bf16)
    adt = A_down.transpose(2, 0, 1).reshape(_F, _ER).astype(bf16)
    wd = W_down.astype(bf16)
    bd = B_down.transpose(0, 2, 1).reshape(_ER, _D).astype(bf16)

    n_tiles = _T // _TILE
    const = lambda i: (0, 0)
    row = lambda i: (i, 0)
    return pl.pallas_call(
        _fused,
        grid=(n_tiles,),
        in_specs=[
            pl.BlockSpec((_TILE, _D), row),
            pl.BlockSpec((_TILE, _D), row),
            pl.BlockSpec((_D, _E), const),
            pl.BlockSpec((_D, 2 * _F), const),
            pl.BlockSpec((_D, 2 * _ER), const),
            pl.BlockSpec((_ER, _F), const),
            pl.BlockSpec((_ER, _F), const),
            pl.BlockSpec((_F, _ER), const),
            pl.BlockSpec((_F, _D), const),
            pl.BlockSpec((_ER, _D), const),
        ],
        out_specs=pl.BlockSpec((_TILE, _D), row),
        out_shape=jax.ShapeDtypeStruct((_T, _D), f32),
    )(hidden_states, xb, W_router, wgu, agu, bg, bu, adt, wd, bd)


# speedup vs baseline: 4.4024x; 1.0015x over previous
import jax
import jax.numpy as jnp
from jax.experimental import pallas as pl

_E = 8
_K = 2
_D = 1024
_F = 2816
_R = 16
_T = 4096
_ER = _E * _R
_TILE = 256


def _fused(x_ref, wr_ref, wg_ref, wu_ref, ag_ref, au_ref, bg_ref, bu_ref,
           adt_ref, wd_ref, bd_ref, out_ref):
    f32 = jnp.float32
    x = x_ref[...]

    logits = jnp.dot(x, wr_ref[...], preferred_element_type=f32)
    eidx = jax.lax.broadcasted_iota(jnp.int32, (_TILE, _E), 1)
    l0 = jnp.max(logits, axis=-1, keepdims=True)
    i0 = jnp.min(jnp.where(logits == l0, eidx, _E), axis=-1, keepdims=True)
    masked = jnp.where(eidx == i0, -jnp.inf, logits)
    l1 = jnp.max(masked, axis=-1, keepdims=True)
    i1 = jnp.min(jnp.where(masked == l1, eidx, _E), axis=-1, keepdims=True)
    w0 = jax.nn.sigmoid(l0 - l1)
    w1 = 1.0 - w0

    g0 = jnp.dot(x, wg_ref[...], preferred_element_type=f32)
    u0 = jnp.dot(x, wu_ref[...], preferred_element_type=f32)
    pg = jnp.dot(x, ag_ref[...], preferred_element_type=f32)
    pu = jnp.dot(x, au_ref[...], preferred_element_type=f32)

    cidx = jax.lax.broadcasted_iota(jnp.int32, (_TILE, _ER), 1) // _R
    hbar = jnp.zeros((_TILE, _F), f32)
    q = jnp.zeros((_TILE, _ER), f32)
    for ik, wk in ((i0, w0), (i1, w1)):
        mk = cidx == ik
        g = g0 + jnp.dot(jnp.where(mk, pg, 0.0), bg_ref[...],
                         preferred_element_type=f32)
        u = u0 + jnp.dot(jnp.where(mk, pu, 0.0), bu_ref[...],
                         preferred_element_type=f32)
        wh = (g * jax.nn.sigmoid(g)) * u * wk
        hbar = hbar + wh
        qf = jnp.dot(wh, adt_ref[...], preferred_element_type=f32)
        q = q + jnp.where(mk, qf, 0.0)

    out_ref[...] = (
        jnp.dot(hbar, wd_ref[...], preferred_element_type=f32)
        + jnp.dot(q, bd_ref[...], preferred_element_type=f32))


def kernel(hidden_states, W_router, W_gate, W_up, W_down,
           A_gate, B_gate, A_up, B_up, A_down, B_down):
    f32 = jnp.float32
    ag = A_gate.reshape(_ER, _D).T
    au = A_up.reshape(_ER, _D).T
    bg = B_gate.transpose(0, 2, 1).reshape(_ER, _F)
    bu = B_up.transpose(0, 2, 1).reshape(_ER, _F)
    adt = A_down.transpose(2, 0, 1).reshape(_F, _ER)
    bd = B_down.transpose(0, 2, 1).reshape(_ER, _D)

    n_tiles = _T // _TILE
    const = lambda i: (0, 0)
    row = lambda i: (i, 0)
    return pl.pallas_call(
        _fused,
        grid=(n_tiles,),
        in_specs=[
            pl.BlockSpec((_TILE, _D), row),
            pl.BlockSpec((_D, _E), const),
            pl.BlockSpec((_D, _F), const),
            pl.BlockSpec((_D, _F), const),
            pl.BlockSpec((_D, _ER), const),
            pl.BlockSpec((_D, _ER), const),
            pl.BlockSpec((_ER, _F), const),
            pl.BlockSpec((_ER, _F), const),
            pl.BlockSpec((_F, _ER), const),
            pl.BlockSpec((_F, _D), const),
            pl.BlockSpec((_ER, _D), const),
        ],
        out_specs=pl.BlockSpec((_TILE, _D), row),
        out_shape=jax.ShapeDtypeStruct((_T, _D), f32),
    )(hidden_states, W_router, W_gate, W_up, ag, au, bg, bu, adt, W_down, bd)


# speedup vs baseline: 5.3217x; 1.2088x over previous
import jax
import jax.numpy as jnp
from jax.experimental import pallas as pl

_E = 8
_K = 2
_D = 1024
_F = 2816
_R = 16
_T = 4096
_ER = _E * _R
_TILE = 256


def _stage1(x_ref, wr_ref, wg_ref, wu_ref, ag_ref, bg_ref, au_ref, bu_ref,
            adt_ref, hbar_ref, q_ref):
    x = x_ref[...]

    logits = jnp.dot(x, wr_ref[...], preferred_element_type=jnp.float32)
    eidx = jax.lax.broadcasted_iota(jnp.int32, (_TILE, _E), 1)
    l0 = jnp.max(logits, axis=-1, keepdims=True)
    i0 = jnp.min(jnp.where(logits == l0, eidx, _E), axis=-1, keepdims=True)
    masked = jnp.where(eidx == i0, -jnp.inf, logits)
    l1 = jnp.max(masked, axis=-1, keepdims=True)
    i1 = jnp.min(jnp.where(masked == l1, eidx, _E), axis=-1, keepdims=True)
    w0 = jax.nn.sigmoid(l0 - l1)
    w1 = 1.0 - w0

    g0 = jnp.dot(x, wg_ref[...], preferred_element_type=jnp.float32)
    u0 = jnp.dot(x, wu_ref[...], preferred_element_type=jnp.float32)
    pg = jnp.dot(x, ag_ref[...], preferred_element_type=jnp.float32)
    pu = jnp.dot(x, au_ref[...], preferred_element_type=jnp.float32)

    cidx = jax.lax.broadcasted_iota(jnp.int32, (_TILE, _ER), 1) // _R
    hbar = jnp.zeros((_TILE, _F), jnp.float32)
    q = jnp.zeros((_TILE, _ER), jnp.float32)
    for ik, wk in ((i0, w0), (i1, w1)):
        mk = cidx == ik
        g = g0 + jnp.dot(jnp.where(mk, pg, 0.0), bg_ref[...],
                         preferred_element_type=jnp.float32)
        u = u0 + jnp.dot(jnp.where(mk, pu, 0.0), bu_ref[...],
                         preferred_element_type=jnp.float32)
        wh = (g * jax.nn.sigmoid(g)) * u * wk
        hbar = hbar + wh
        qf = jnp.dot(wh, adt_ref[...], preferred_element_type=jnp.float32)
        q = q + jnp.where(mk, qf, 0.0)
    hbar_ref[...] = hbar
    q_ref[...] = q


def _stage2(hbar_ref, q_ref, wd_ref, bd_ref, out_ref):
    out_ref[...] = (
        jnp.dot(hbar_ref[...], wd_ref[...], preferred_element_type=jnp.float32)
        + jnp.dot(q_ref[...], bd_ref[...], preferred_element_type=jnp.float32))


def kernel(hidden_states, W_router, W_gate, W_up, W_down,
           A_gate, B_gate, A_up, B_up, A_down, B_down):
    f32 = jnp.float32
    ag = A_gate.reshape(_ER, _D).T
    au = A_up.reshape(_ER, _D).T
    bg = B_gate.transpose(0, 2, 1).reshape(_ER, _F)
    bu = B_up.transpose(0, 2, 1).reshape(_ER, _F)
    adt = A_down.transpose(2, 0, 1).reshape(_F, _ER)
    bd = B_down.transpose(0, 2, 1).reshape(_ER, _D)

    n_tiles = _T // _TILE
    const = lambda i: (0, 0)
    row = lambda i: (i, 0)
    hbar, q = pl.pallas_call(
        _stage1,
        grid=(n_tiles,),
        in_specs=[
            pl.BlockSpec((_TILE, _D), row),
            pl.BlockSpec((_D, _E), const),
            pl.BlockSpec((_D, _F), const),
            pl.BlockSpec((_D, _F), const),
            pl.BlockSpec((_D, _ER), const),
            pl.BlockSpec((_ER, _F), const),
            pl.BlockSpec((_D, _ER), const),
            pl.BlockSpec((_ER, _F), const),
            pl.BlockSpec((_F, _ER), const),
        ],
        out_specs=[
            pl.BlockSpec((_TILE, _F), row),
            pl.BlockSpec((_TILE, _ER), row),
        ],
        out_shape=[
            jax.ShapeDtypeStruct((_T, _F), f32),
            jax.ShapeDtypeStruct((_T, _ER), f32),
        ],
    )(hidden_states, W_router, W_gate, W_up, ag, bg, au, bu, adt)

    out = pl.pallas_call(
        _stage2,
        grid=(n_tiles,),
        in_specs=[
            pl.BlockSpec((_TILE, _F), row),
            pl.BlockSpec((_TILE, _ER), row),
            pl.BlockSpec((_F, _D), const),
            pl.BlockSpec((_ER, _D), const),
        ],
        out_specs=pl.BlockSpec((_TILE, _D), row),
        out_shape=jax.ShapeDtypeStruct((_T, _D), f32),
    )(hbar, q, W_down, bd)
    return out


# speedup vs baseline: 5.6299x; 1.0579x over previous
import jax
import jax.numpy as jnp
from jax.experimental import pallas as pl

_E = 8
_K = 2
_D = 1024
_F = 2816
_R = 16
_T = 4096
_ER = _E * _R
_TILE = 256
_TILE2 = 1024
_HALF = 256


def _mix_half(x, wr, wg, wu, ag, au, bg, bu, adt):
    f32 = jnp.float32
    m = x.shape[0]

    logits = jnp.dot(x, wr, preferred_element_type=f32)
    eidx = jax.lax.broadcasted_iota(jnp.int32, (m, _E), 1)
    l0 = jnp.max(logits, axis=-1, keepdims=True)
    i0 = jnp.min(jnp.where(logits == l0, eidx, _E), axis=-1, keepdims=True)
    masked = jnp.where(eidx == i0, -jnp.inf, logits)
    l1 = jnp.max(masked, axis=-1, keepdims=True)
    i1 = jnp.min(jnp.where(masked == l1, eidx, _E), axis=-1, keepdims=True)
    w0 = jax.nn.sigmoid(l0 - l1)
    w1 = 1.0 - w0

    g0 = jnp.dot(x, wg, preferred_element_type=f32)
    u0 = jnp.dot(x, wu, preferred_element_type=f32)
    dn_rt = (((1,), (1,)), ((), ()))
    pg = jax.lax.dot_general(x, ag, dn_rt, preferred_element_type=f32)
    pu = jax.lax.dot_general(x, au, dn_rt, preferred_element_type=f32)

    cidx = jax.lax.broadcasted_iota(jnp.int32, (m, _ER), 1) // _R
    hbar = jnp.zeros((m, _F), f32)
    q = jnp.zeros((m, _ER), f32)
    for ik, wk in ((i0, w0), (i1, w1)):
        mk = cidx == ik
        g = g0 + jnp.dot(jnp.where(mk, pg, 0.0), bg, preferred_element_type=f32)
        u = u0 + jnp.dot(jnp.where(mk, pu, 0.0), bu, preferred_element_type=f32)
        wh = (g * jax.nn.sigmoid(g)) * u * wk
        hbar = hbar + wh
        qf = jax.lax.dot_general(wh, adt, (((1,), (1,)), ((), ())),
                                 preferred_element_type=f32)
        q = q + jnp.where(mk, qf, 0.0)
    return hbar, q


def _stage1(x_ref, wr_ref, wg_ref, wu_ref, ag_ref, bg_ref, au_ref, bu_ref,
            adt_ref, hbar_ref, q_ref):
    wr = wr_ref[...]
    wg = wg_ref[...]
    wu = wu_ref[...]
    ag = ag_ref[...]
    au = au_ref[...]
    bg = bg_ref[...]
    bu = bu_ref[...]
    adt = adt_ref[...]
    for h in range(_TILE // _HALF):
        sl = pl.ds(h * _HALF, _HALF)
        hbar, q = _mix_half(x_ref[sl, :], wr, wg, wu, ag, au, bg, bu, adt)
        hbar_ref[sl, :] = hbar
        q_ref[sl, :] = q


def _stage2(hbar_ref, q_ref, wd_ref, bd_ref, out_ref):
    out_ref[...] = (
        jnp.dot(hbar_ref[...], wd_ref[...], preferred_element_type=jnp.float32)
        + jnp.dot(q_ref[...], bd_ref[...], preferred_element_type=jnp.float32))


def kernel(hidden_states, W_router, W_gate, W_up, W_down,
           A_gate, B_gate, A_up, B_up, A_down, B_down):
    f32 = jnp.float32
    ag = A_gate.reshape(_ER, _D)
    au = A_up.reshape(_ER, _D)
    bg = B_gate.transpose(0, 2, 1).reshape(_ER, _F)
    bu = B_up.transpose(0, 2, 1).reshape(_ER, _F)
    adt = A_down.reshape(_ER, _F)
    bd = B_down.transpose(0, 2, 1).reshape(_ER, _D)

    n_tiles = _T // _TILE
    const = lambda i: (0, 0)
    row = lambda i: (i, 0)
    hbar, q = pl.pallas_call(
        _stage1,
        grid=(n_tiles,),
        in_specs=[
            pl.BlockSpec((_TILE, _D), row),
            pl.BlockSpec((_D, _E), const),
            pl.BlockSpec((_D, _F), const),
            pl.BlockSpec((_D, _F), const),
            pl.BlockSpec((_ER, _D), const),
            pl.BlockSpec((_ER, _F), const),
            pl.BlockSpec((_ER, _D), const),
            pl.BlockSpec((_ER, _F), const),
            pl.BlockSpec((_ER, _F), const),
        ],
        out_specs=[
            pl.BlockSpec((_TILE, _F), row),
            pl.BlockSpec((_TILE, _ER), row),
        ],
        out_shape=[
            jax.ShapeDtypeStruct((_T, _F), f32),
            jax.ShapeDtypeStruct((_T, _ER), f32),
        ],
    )(hidden_states, W_router, W_gate, W_up, ag, bg, au, bu, adt)

    out = pl.pallas_call(
        _stage2,
        grid=(_T // _TILE2,),
        in_specs=[
            pl.BlockSpec((_TILE2, _F), row),
            pl.BlockSpec((_TILE2, _ER), row),
            pl.BlockSpec((_F, _D), const),
            pl.BlockSpec((_ER, _D), const),
        ],
        out_specs=pl.BlockSpec((_TILE2, _D), row),
        out_shape=jax.ShapeDtypeStruct((_T, _D), f32),
    )(hbar, q, W_down, bd)
    return out


# speedup vs baseline: 5.7126x; 1.0147x over previous
import jax
import jax.numpy as jnp
from jax.experimental import pallas as pl
from jax.experimental.pallas import tpu as pltpu

_E = 8
_K = 2
_D = 1024
_F = 2816
_R = 16
_T = 4096
_ER = _E * _R
_TILE = 256
_NT = _T // _TILE


def _mix_tile(x, wr, wg, wu, ag, au, bg, bu, adt):
    f32 = jnp.float32
    m = x.shape[0]

    logits = jnp.dot(x, wr, preferred_element_type=f32)
    eidx = jax.lax.broadcasted_iota(jnp.int32, (m, _E), 1)
    l0 = jnp.max(logits, axis=-1, keepdims=True)
    i0 = jnp.min(jnp.where(logits == l0, eidx, _E), axis=-1, keepdims=True)
    masked = jnp.where(eidx == i0, -jnp.inf, logits)
    l1 = jnp.max(masked, axis=-1, keepdims=True)
    i1 = jnp.min(jnp.where(masked == l1, eidx, _E), axis=-1, keepdims=True)
    w0 = jax.nn.sigmoid(l0 - l1)
    w1 = 1.0 - w0

    g0 = jnp.dot(x, wg, preferred_element_type=f32)
    u0 = jnp.dot(x, wu, preferred_element_type=f32)
    dn_rt = (((1,), (1,)), ((), ()))
    pg = jax.lax.dot_general(x, ag, dn_rt, preferred_element_type=f32)
    pu = jax.lax.dot_general(x, au, dn_rt, preferred_element_type=f32)

    cidx = jax.lax.broadcasted_iota(jnp.int32, (m, _ER), 1) // _R
    hbar = jnp.zeros((m, _F), f32)
    q = jnp.zeros((m, _ER), f32)
    for ik, wk in ((i0, w0), (i1, w1)):
        mk = cidx == ik
        g = g0 + jnp.dot(jnp.where(mk, pg, 0.0), bg, preferred_element_type=f32)
        u = u0 + jnp.dot(jnp.where(mk, pu, 0.0), bu, preferred_element_type=f32)
        wh = (g * jax.nn.sigmoid(g)) * u * wk
        hbar = hbar + wh
        qf = jax.lax.dot_general(wh, adt, (((1,), (1,)), ((), ())),
                                 preferred_element_type=f32)
        q = q + jnp.where(mk, qf, 0.0)
    return hbar, q


def _fused(x_ref, wr_ref, wg_ref, wu_ref, ag_ref, bg_ref, au_ref, bu_ref,
           adt_ref, wd_ref, bd_ref, out_ref, hb_ref, qq_ref):
    i = pl.program_id(0)
    cur = jax.lax.rem(i, 2)
    prev = 1 - cur

    out_ref[...] = (
        jnp.dot(hb_ref[prev], wd_ref[...], preferred_element_type=jnp.float32)
        + jnp.dot(qq_ref[prev], bd_ref[...], preferred_element_type=jnp.float32))

    hbar, q = _mix_tile(x_ref[...], wr_ref[...], wg_ref[...], wu_ref[...],
                        ag_ref[...], au_ref[...], bg_ref[...], bu_ref[...],
                        adt_ref[...])
    hb_ref[cur] = hbar.astype(jnp.bfloat16)
    qq_ref[cur] = q.astype(jnp.bfloat16)


def kernel(hidden_states, W_router, W_gate, W_up, W_down,
           A_gate, B_gate, A_up, B_up, A_down, B_down):
    f32 = jnp.float32
    ag = A_gate.reshape(_ER, _D)
    au = A_up.reshape(_ER, _D)
    bg = B_gate.transpose(0, 2, 1).reshape(_ER, _F)
    bu = B_up.transpose(0, 2, 1).reshape(_ER, _F)
    adt = A_down.reshape(_ER, _F)
    bd = B_down.transpose(0, 2, 1).reshape(_ER, _D)

    const = lambda i: (0, 0)
    return pl.pallas_call(
        _fused,
        grid=(_NT + 1,),
        in_specs=[
            pl.BlockSpec((_TILE, _D), lambda i: (jnp.minimum(i, _NT - 1), 0)),
            pl.BlockSpec((_D, _E), const),
            pl.BlockSpec((_D, _F), const),
            pl.BlockSpec((_D, _F), const),
            pl.BlockSpec((_ER, _D), const),
            pl.BlockSpec((_ER, _F), const),
            pl.BlockSpec((_ER, _D), const),
            pl.BlockSpec((_ER, _F), const),
            pl.BlockSpec((_ER, _F), const),
            pl.BlockSpec((_F, _D), const),
            pl.BlockSpec((_ER, _D), const),
        ],
        out_specs=pl.BlockSpec((_TILE, _D), lambda i: (jnp.maximum(i - 1, 0), 0)),
        out_shape=jax.ShapeDtypeStruct((_T, _D), f32),
        scratch_shapes=[
            pltpu.VMEM((2, _TILE, _F), jnp.bfloat16),
            pltpu.VMEM((2, _TILE, _ER), jnp.bfloat16),
        ],
    )(hidden_states, W_router, W_gate, W_up, ag, bg, au, bu, adt, W_down, bd)


# speedup vs baseline: 5.9166x; 1.0357x over previous
import jax
import jax.numpy as jnp
from jax.experimental import pallas as pl
from jax.experimental.pallas import tpu as pltpu

_E = 8
_K = 2
_D = 1024
_F = 2816
_R = 16
_T = 4096
_ER = _E * _R
_TILE = 256
_NT = _T // _TILE


def _mix_tile(x, wr, wg, wu, ag, au, bg, bu, adt):
    f32 = jnp.float32
    m = x.shape[0]

    logits = jnp.dot(x, wr, preferred_element_type=f32)
    eidx = jax.lax.broadcasted_iota(jnp.int32, (m, _E), 1)
    l0 = jnp.max(logits, axis=-1, keepdims=True)
    i0 = jnp.min(jnp.where(logits == l0, eidx, _E), axis=-1, keepdims=True)
    masked = jnp.where(eidx == i0, -jnp.inf, logits)
    l1 = jnp.max(masked, axis=-1, keepdims=True)
    i1 = jnp.min(jnp.where(masked == l1, eidx, _E), axis=-1, keepdims=True)
    w0 = jax.nn.sigmoid(l0 - l1)
    w1 = 1.0 - w0

    g0 = jnp.dot(x, wg, preferred_element_type=f32)
    u0 = jnp.dot(x, wu, preferred_element_type=f32)
    dn_rt = (((1,), (1,)), ((), ()))
    pg = jax.lax.dot_general(x, ag, dn_rt, preferred_element_type=f32)
    pu = jax.lax.dot_general(x, au, dn_rt, preferred_element_type=f32)

    cidx = jax.lax.broadcasted_iota(jnp.int32, (m, _ER), 1) // _R
    hbar = jnp.zeros((m, _F), f32)
    q = jnp.zeros((m, _ER), f32)
    for ik, wk in ((i0, w0), (i1, w1)):
        mk = cidx == ik
        g = g0 + jnp.dot(jnp.where(mk, pg, 0.0), bg, preferred_element_type=f32)
        u = u0 + jnp.dot(jnp.where(mk, pu, 0.0), bu, preferred_element_type=f32)
        wh = (g * jax.nn.sigmoid(g)) * u * wk
        hbar = hbar + wh
        qf = jax.lax.dot_general(wh, adt, (((1,), (1,)), ((), ())),
                                 preferred_element_type=f32)
        q = q + jnp.where(mk, qf, 0.0)
    return hbar, q


def _fused(x_ref, wr_ref, wg_ref, wu_ref, ag_ref, bg_ref, au_ref, bu_ref,
           adt_ref, wd_ref, bd_ref, out_ref, hb_ref, qq_ref):
    i = pl.program_id(0)
    cur = jax.lax.rem(i, 2)
    prev = 1 - cur

    f32 = jnp.float32
    x = x_ref[...]
    wr = wr_ref[...]
    bg = bg_ref[...]
    bu = bu_ref[...]
    adt = adt_ref[...]
    m = _TILE

    logits = jnp.dot(x, wr, preferred_element_type=f32)
    eidx = jax.lax.broadcasted_iota(jnp.int32, (m, _E), 1)
    l0 = jnp.max(logits, axis=-1, keepdims=True)
    i0 = jnp.min(jnp.where(logits == l0, eidx, _E), axis=-1, keepdims=True)
    masked = jnp.where(eidx == i0, -jnp.inf, logits)
    l1 = jnp.max(masked, axis=-1, keepdims=True)
    i1 = jnp.min(jnp.where(masked == l1, eidx, _E), axis=-1, keepdims=True)
    w0 = jax.nn.sigmoid(l0 - l1)
    w1 = 1.0 - w0

    g0 = jnp.dot(x, wg_ref[...], preferred_element_type=f32)
    u0 = jnp.dot(x, wu_ref[...], preferred_element_type=f32)
    dn_rt = (((1,), (1,)), ((), ()))
    pg = jax.lax.dot_general(x, ag_ref[...], dn_rt, preferred_element_type=f32)
    pu = jax.lax.dot_general(x, au_ref[...], dn_rt, preferred_element_type=f32)

    out_ref[...] = (
        jnp.dot(hb_ref[prev], wd_ref[...], preferred_element_type=jnp.float32)
        + jnp.dot(qq_ref[prev], bd_ref[...], preferred_element_type=jnp.float32))

    cidx = jax.lax.broadcasted_iota(jnp.int32, (m, _ER), 1) // _R
    hbar = jnp.zeros((m, _F), f32)
    q = jnp.zeros((m, _ER), f32)
    for ik, wk in ((i0, w0), (i1, w1)):
        mk = cidx == ik
        g = g0 + jnp.dot(jnp.where(mk, pg, 0.0), bg, preferred_element_type=f32)
        u = u0 + jnp.dot(jnp.where(mk, pu, 0.0), bu, preferred_element_type=f32)
        wh = (g * jax.nn.sigmoid(g)) * u * wk
        hbar = hbar + wh
        qf = jax.lax.dot_general(wh, adt, (((1,), (1,)), ((), ())),
                                 preferred_element_type=f32)
        q = q + jnp.where(mk, qf, 0.0)
    hb_ref[cur] = hbar.astype(jnp.bfloat16)
    qq_ref[cur] = q.astype(jnp.bfloat16)


def kernel(hidden_states, W_router, W_gate, W_up, W_down,
           A_gate, B_gate, A_up, B_up, A_down, B_down):
    f32 = jnp.float32
    ag = A_gate.reshape(_ER, _D)
    au = A_up.reshape(_ER, _D)
    bg = B_gate.transpose(0, 2, 1).reshape(_ER, _F)
    bu = B_up.transpose(0, 2, 1).reshape(_ER, _F)
    adt = A_down.reshape(_ER, _F)
    bd = B_down.transpose(0, 2, 1).reshape(_ER, _D)

    const = lambda i: (0, 0)
    return pl.pallas_call(
        _fused,
        grid=(_NT + 1,),
        in_specs=[
            pl.BlockSpec((_TILE, _D), lambda i: (jnp.minimum(i, _NT - 1), 0)),
            pl.BlockSpec((_D, _E), const),
            pl.BlockSpec((_D, _F), const),
            pl.BlockSpec((_D, _F), const),
            pl.BlockSpec((_ER, _D), const),
            pl.BlockSpec((_ER, _F), const),
            pl.BlockSpec((_ER, _D), const),
            pl.BlockSpec((_ER, _F), const),
            pl.BlockSpec((_ER, _F), const),
            pl.BlockSpec((_F, _D), const),
            pl.BlockSpec((_ER, _D), const),
        ],
        out_specs=pl.BlockSpec((_TILE, _D), lambda i: (jnp.maximum(i - 1, 0), 0)),
        out_shape=jax.ShapeDtypeStruct((_T, _D), f32),
        scratch_shapes=[
            pltpu.VMEM((2, _TILE, _F), jnp.bfloat16),
            pltpu.VMEM((2, _TILE, _ER), jnp.bfloat16),
        ],
    )(hidden_states, W_router, W_gate, W_up, ag, bg, au, bu, adt, W_down, bd)
